# baseline (speedup 1.0000x reference)
import functools

import jax
import jax.numpy as jnp
import numpy as np
from jax import lax
from jax.experimental import pallas as pl
from jax.experimental.pallas import tpu as pltpu

N = 10000
E = 160000
NG = 32
EPS = 1e-5
H, C = 8, 48
D = 384
BN = 1000
BE = 1000
RSQRT_C = 1.0 / np.sqrt(48.0)

_f32 = jnp.float32


def _cdiv(a, b):
    return (a + b - 1) // b


def _enc_node_body(xv_ref, xg_ref, xp_ref, Wv_ref, bv_ref, Wg_ref, bg_ref,
                   Wp_ref, bp_ref, z_ref, st_ref, acc):
    zv = jnp.maximum(jnp.dot(xv_ref[...], Wv_ref[...],
                             preferred_element_type=_f32) + bv_ref[...], 0.0)
    zg = jnp.maximum(jnp.dot(xg_ref[...], Wg_ref[...],
                             preferred_element_type=_f32) + bg_ref[...], 0.0)
    zp = jnp.maximum(jnp.dot(xp_ref[...], Wp_ref[...],
                             preferred_element_type=_f32) + bp_ref[...], 0.0)
    z_ref[...] = jnp.concatenate([zv, zg, zp], axis=1)
    vals = jnp.concatenate(
        [zv.sum(1, keepdims=True), (zv * zv).sum(1, keepdims=True),
         zg.sum(1, keepdims=True), (zg * zg).sum(1, keepdims=True),
         zp.sum(1, keepdims=True), (zp * zp).sum(1, keepdims=True)], axis=1)
    ones = jnp.full((8, vals.shape[0]), 1.0, _f32)
    part = jnp.dot(ones, vals, preferred_element_type=_f32)

    @pl.when(pl.program_id(0) == 0)
    def _():
        acc[...] = jnp.zeros_like(acc)

    acc[...] += part

    @pl.when(pl.program_id(0) == pl.num_programs(0) - 1)
    def _():
        st_ref[...] = acc[...]


def _encode_nodes(xv, xg, xp, p):
    grid = N // BN
    return pl.pallas_call(
        _enc_node_body,
        grid=(grid,),
        in_specs=[
            pl.BlockSpec((BN, 1024), lambda i: (i, 0)),
            pl.BlockSpec((BN, 6), lambda i: (i, 0)),
            pl.BlockSpec((BN, 50), lambda i: (i, 0)),
            pl.BlockSpec((1024, 128), lambda i: (0, 0)),
            pl.BlockSpec((1, 128), lambda i: (0, 0)),
            pl.BlockSpec((6, 128), lambda i: (0, 0)),
            pl.BlockSpec((1, 128), lambda i: (0, 0)),
            pl.BlockSpec((50, 128), lambda i: (0, 0)),
            pl.BlockSpec((1, 128), lambda i: (0, 0)),
        ],
        out_specs=[
            pl.BlockSpec((BN, D), lambda i: (i, 0)),
            pl.BlockSpec((8, 6), lambda i: (0, 0)),
        ],
        out_shape=[
            jax.ShapeDtypeStruct((N, D), _f32),
            jax.ShapeDtypeStruct((8, 6), _f32),
        ],
        scratch_shapes=[pltpu.VMEM((8, 6), _f32)],
    )(xv, xg, xp, p['Wvis'], p['bvis'].reshape(1, 128),
      p['Wg'], p['bg'].reshape(1, 128), p['Wp'], p['bp'].reshape(1, 128))


def _enc_edge_body(xe_ref, W_ref, b_ref, z_ref, st_ref, acc):
    z = jnp.maximum(jnp.dot(xe_ref[...], W_ref[...],
                            preferred_element_type=_f32) + b_ref[...], 0.0)
    z_ref[...] = z
    vals = jnp.concatenate(
        [z.sum(1, keepdims=True), (z * z).sum(1, keepdims=True)], axis=1)
    ones = jnp.full((8, vals.shape[0]), 1.0, _f32)
    part = jnp.dot(ones, vals, preferred_element_type=_f32)

    @pl.when(pl.program_id(0) == 0)
    def _():
        acc[...] = jnp.zeros_like(acc)

    acc[...] += part

    @pl.when(pl.program_id(0) == pl.num_programs(0) - 1)
    def _():
        st_ref[...] = acc[...]


def _encode_edges(xe, p):
    grid = E // BE
    return pl.pallas_call(
        _enc_edge_body,
        grid=(grid,),
        in_specs=[
            pl.BlockSpec((BE, 3), lambda i: (i, 0)),
            pl.BlockSpec((3, 128), lambda i: (0, 0)),
            pl.BlockSpec((1, 128), lambda i: (0, 0)),
        ],
        out_specs=[
            pl.BlockSpec((BE, 128), lambda i: (i, 0)),
            pl.BlockSpec((8, 2), lambda i: (0, 0)),
        ],
        out_shape=[
            jax.ShapeDtypeStruct((E, 128), _f32),
            jax.ShapeDtypeStruct((8, 2), _f32),
        ],
        scratch_shapes=[pltpu.VMEM((8, 2), _f32)],
    )(xe, p['Wee'], p['bee'].reshape(1, 128))


def _affine_body(z_ref, w_ref, b_ref, h_ref):
    h_ref[...] = z_ref[...] * w_ref[...] + b_ref[...]


def _affine(z, w_row, b_row):
    n, d = z.shape
    grid = n // BN
    return pl.pallas_call(
        _affine_body,
        grid=(grid,),
        in_specs=[
            pl.BlockSpec((BN, d), lambda i: (i, 0)),
            pl.BlockSpec((1, d), lambda i: (0, 0)),
            pl.BlockSpec((1, d), lambda i: (0, 0)),
        ],
        out_specs=pl.BlockSpec((BN, d), lambda i: (i, 0)),
        out_shape=jax.ShapeDtypeStruct((n, d), _f32),
    )(z, w_row, b_row)


def _qkv_body(h_ref, Wq_ref, bq_ref, Wk_ref, bk_ref, Wv_ref, bv_ref,
              q_ref, k_ref, v_ref):
    h = h_ref[...]
    z = jnp.zeros((h.shape[0], 128), jnp.bfloat16)
    q = jnp.dot(h, Wq_ref[...], preferred_element_type=_f32) + bq_ref[...]
    q_ref[...] = jnp.concatenate([q.astype(jnp.bfloat16), z], axis=1)
    k = jnp.dot(h, Wk_ref[...], preferred_element_type=_f32) + bk_ref[...]
    k_ref[...] = jnp.concatenate([k.astype(jnp.bfloat16), z], axis=1)
    v = jnp.dot(h, Wv_ref[...], preferred_element_type=_f32) + bv_ref[...]
    v_ref[...] = jnp.concatenate([v.astype(jnp.bfloat16), z], axis=1)


def _qkv(h, p, s):
    grid = N // BN
    w = pl.BlockSpec((D, D), lambda i: (0, 0))
    b = pl.BlockSpec((1, D), lambda i: (0, 0))
    r = pl.BlockSpec((BN, D), lambda i: (i, 0))
    r512 = pl.BlockSpec((BN, 512), lambda i: (i, 0))
    return pl.pallas_call(
        _qkv_body,
        grid=(grid,),
        in_specs=[r, w, b, w, b, w, b],
        out_specs=[r512, r512, r512],
        out_shape=[jax.ShapeDtypeStruct((N, 512), jnp.bfloat16)] * 3,
    )(h, p['Wq' + s], p['bq' + s].reshape(1, D),
      p['Wk' + s], p['bk' + s].reshape(1, D),
      p['Wval' + s], p['bval' + s].reshape(1, D))


def _edgeproj_body(z_ref, W1_ref, b1_ref, W2_ref, b2_ref, e1_ref, e2_ref):
    z = z_ref[...]
    e1_ref[...] = (jnp.dot(z, W1_ref[...], preferred_element_type=_f32)
                   + b1_ref[...]).astype(jnp.bfloat16)
    e2_ref[...] = (jnp.dot(z, W2_ref[...], preferred_element_type=_f32)
                   + b2_ref[...]).astype(jnp.bfloat16)


def _edgeproj(ze, W1, b1, W2, b2):
    grid = E // BE
    return pl.pallas_call(
        _edgeproj_body,
        grid=(grid,),
        in_specs=[
            pl.BlockSpec((BE, 128), lambda i: (i, 0)),
            pl.BlockSpec((128, D), lambda i: (0, 0)),
            pl.BlockSpec((1, D), lambda i: (0, 0)),
            pl.BlockSpec((128, D), lambda i: (0, 0)),
            pl.BlockSpec((1, D), lambda i: (0, 0)),
        ],
        out_specs=[pl.BlockSpec((BE, D), lambda i: (i, 0))] * 2,
        out_shape=[jax.ShapeDtypeStruct((E, D), jnp.bfloat16)] * 2,
    )(ze, W1, b1.reshape(1, D), W2, b2.reshape(1, D))


def _edge_body(qd_ref, ks_ref, vs_ref, e_ref, hsel_ref, hd_ref,
               ex_ref, ma_ref, mb_ref, mc_ref):
    e = e_ref[...].astype(_f32)
    qd = qd_ref[:, :D].astype(_f32)
    ks = ks_ref[:, :D].astype(_f32)
    vs = vs_ref[:, :D].astype(_f32)
    t = qd * (ks + e)
    a = jnp.dot(t, hsel_ref[...], preferred_element_type=_f32) * RSQRT_C
    mask = (lax.broadcasted_iota(jnp.int32, a.shape, 1) < H).astype(_f32)
    ex = jnp.exp(a) * mask
    ex_ref[...] = ex
    a384 = jnp.dot(ex, hd_ref[...], preferred_element_type=_f32)
    m = (vs + e) * a384
    ma_ref[...] = m[:, :128]
    mb_ref[...] = m[:, 128:256]
    mc_ref[...] = m[:, 256:]


def _edge_attn(qd, ks, vs, e, hsel128, hd):
    grid = E // BE
    r = pl.BlockSpec((BE, D), lambda i: (i, 0))
    r512 = pl.BlockSpec((BE, 512), lambda i: (i, 0))
    r128 = pl.BlockSpec((BE, 128), lambda i: (i, 0))
    return pl.pallas_call(
        _edge_body,
        grid=(grid,),
        in_specs=[r512, r512, r512, r,
                  pl.BlockSpec((D, 128), lambda i: (0, 0)),
                  pl.BlockSpec((128, D), lambda i: (0, 0))],
        out_specs=[r128] * 4,
        out_shape=[jax.ShapeDtypeStruct((E, 128), _f32)] * 4,
    )(qd, ks, vs, e, hsel128, hd)


def _combine_body(h_ref, a_ref, b_ref, c0_ref, c1_ref, d0_ref, d1_ref,
                  hd_ref, Ws_ref, bs_ref, oh_ref, t_ref, st_ref, acc):
    h = h_ref[...]
    den = d0_ref[...] + d1_ref[...]
    den_exp = jnp.dot(den, hd_ref[...], preferred_element_type=_f32)
    o = jnp.concatenate([a_ref[...], b_ref[...], c0_ref[...] + c1_ref[...]],
                        axis=1)
    o = o / (den_exp + 1e-16)
    o = o + jnp.dot(h, Ws_ref[...], preferred_element_type=_f32) + bs_ref[...]
    t = h + jnp.where(o > 0, o, jnp.exp(jnp.minimum(o, 0.0)) - 1.0)
    t_ref[...] = t
    rs = t.sum(1, keepdims=True)
    rq = (t * t).sum(1, keepdims=True)
    vals = jnp.concatenate([rs, rq, jnp.full_like(rs, float(D))], axis=1)
    part = lax.dot_general(oh_ref[...], vals, (((0,), (0,)), ((), ())),
                           preferred_element_type=_f32)

    @pl.when(pl.program_id(0) == 0)
    def _():
        acc[...] = jnp.zeros_like(acc)

    acc[...] += part

    @pl.when(pl.program_id(0) == pl.num_programs(0) - 1)
    def _():
        st_ref[...] = acc[...]


def _combine(h, msg_full, msg_parts, den_parts, hd, Ws, bs, oh):
    grid = N // BN
    r128 = pl.BlockSpec((BN, 128), lambda i: (i, 0))
    return pl.pallas_call(
        _combine_body,
        grid=(grid,),
        in_specs=[
            pl.BlockSpec((BN, D), lambda i: (i, 0)),
            r128, r128, r128, r128, r128, r128,
            pl.BlockSpec((128, D), lambda i: (0, 0)),
            pl.BlockSpec((D, D), lambda i: (0, 0)),
            pl.BlockSpec((1, D), lambda i: (0, 0)),
            r128,
        ],
        out_specs=[
            pl.BlockSpec((BN, D), lambda i: (i, 0)),
            pl.BlockSpec((128, 3), lambda i: (0, 0)),
        ],
        out_shape=[
            jax.ShapeDtypeStruct((N, D), _f32),
            jax.ShapeDtypeStruct((128, 3), _f32),
        ],
        scratch_shapes=[pltpu.VMEM((128, 3), _f32)],
    )(h, msg_full[0, :N], msg_full[1, :N], msg_parts[0, :N], msg_parts[1, :N],
      den_parts[0, :N], den_parts[1, :N], hd, Ws, bs.reshape(1, D), oh)


def _bnorm_body(t_ref, oh_ref, mv_ref, w_ref, b_ref, h_ref):
    mg = jnp.dot(oh_ref[...], mv_ref[...], preferred_element_type=_f32)
    m = mg[:, 0:1]
    r = mg[:, 1:2]
    h_ref[...] = (t_ref[...] - m) * r * w_ref[...] + b_ref[...]


def _bnorm(t, oh, mv, w, b):
    grid = N // BN
    return pl.pallas_call(
        _bnorm_body,
        grid=(grid,),
        in_specs=[
            pl.BlockSpec((BN, D), lambda i: (i, 0)),
            pl.BlockSpec((BN, 128), lambda i: (i, 0)),
            pl.BlockSpec((128, 2), lambda i: (0, 0)),
            pl.BlockSpec((1, D), lambda i: (0, 0)),
            pl.BlockSpec((1, D), lambda i: (0, 0)),
        ],
        out_specs=pl.BlockSpec((BN, D), lambda i: (i, 0)),
        out_shape=jax.ShapeDtypeStruct((N, D), _f32),
    )(t, oh, mv, w.reshape(1, D), b.reshape(1, D))


def _final_body(h_ref, W1_ref, b1_ref, W2_ref, b2_ref, o_ref):
    l1 = jnp.maximum(jnp.dot(h_ref[...], W1_ref[...],
                             preferred_element_type=_f32) + b1_ref[...], 0.0)
    o_ref[...] = jnp.dot(l1, W2_ref[...], preferred_element_type=_f32) + b2_ref[...]


def _final(h2, W1, b1, W2p, b2p):
    grid = N // BN
    return pl.pallas_call(
        _final_body,
        grid=(grid,),
        in_specs=[
            pl.BlockSpec((BN, D), lambda i: (i, 0)),
            pl.BlockSpec((D, 128), lambda i: (0, 0)),
            pl.BlockSpec((1, 128), lambda i: (0, 0)),
            pl.BlockSpec((128, 128), lambda i: (0, 0)),
            pl.BlockSpec((1, 128), lambda i: (0, 0)),
        ],
        out_specs=pl.BlockSpec((BN, 128), lambda i: (i, 0)),
        out_shape=jax.ShapeDtypeStruct((N, 128), _f32),
    )(h2, W1, b1.reshape(1, 128), W2p, b2p.reshape(1, 128))


from jax.experimental.pallas import tpu_sc as plsc

NW = 32
CH = 40


def _mesh():
    return plsc.VectorSubcoreMesh(core_axis_name="c", subcore_axis_name="s")


def _gather_qk(q, k, dst, src):
    per = E // NW
    CHG = 128
    nch = per // CHG
    TL = per - nch * CHG
    @functools.partial(
        pl.kernel,
        out_type=[jax.ShapeDtypeStruct((E, 256), jnp.int32)] * 2,
        mesh=_mesh(),
        scratch_types=[
            pltpu.VMEM((CHG,), jnp.int32),
            pltpu.VMEM((CHG,), jnp.int32),
            pltpu.VMEM((CHG, 256), jnp.int32),
            pltpu.VMEM((CHG, 256), jnp.int32),
            pltpu.SemaphoreType.DMA,
            pltpu.SemaphoreType.DMA,
        ],
    )
    def run(q_hbm, k_hbm, dst_hbm, src_hbm, qd_hbm, ks_hbm,
            idxd, idxs, qrows, krows, sem1, sem2):
        wid = lax.axis_index("s") * 2 + lax.axis_index("c")
        base0 = wid * per

        def chunk(base, n):
            ci = pltpu.async_copy(dst_hbm.at[pl.ds(base, n)],
                                  idxd.at[pl.ds(0, n)], sem1)
            cj = pltpu.async_copy(src_hbm.at[pl.ds(base, n)],
                                  idxs.at[pl.ds(0, n)], sem2)
            ci.wait()
            cj.wait()
            cq = pltpu.async_copy(q_hbm.at[idxd.at[pl.ds(0, n)]],
                                  qrows.at[pl.ds(0, n)], sem1)
            ck = pltpu.async_copy(k_hbm.at[idxs.at[pl.ds(0, n)]],
                                  krows.at[pl.ds(0, n)], sem2)
            cq.wait()
            ck.wait()
            wq = pltpu.async_copy(qrows.at[pl.ds(0, n)],
                                  qd_hbm.at[pl.ds(base, n)], sem1)
            wk = pltpu.async_copy(krows.at[pl.ds(0, n)],
                                  ks_hbm.at[pl.ds(base, n)], sem2)
            wq.wait()
            wk.wait()

        def step(i, carry):
            chunk(base0 + i * CHG, CHG)
            return carry

        lax.fori_loop(0, nch, step, 0)
        chunk(base0 + nch * CHG, TL)

    return run(q, k, dst, src)


def _gather_v(v, src):
    per = E // NW
    CHG = 128
    nch = per // CHG
    TL = per - nch * CHG
    npair = nch // 2

    @functools.partial(
        pl.kernel,
        out_type=jax.ShapeDtypeStruct((E, 256), jnp.int32),
        mesh=_mesh(),
        scratch_types=[
            pltpu.VMEM((CHG,), jnp.int32),
            pltpu.VMEM((CHG,), jnp.int32),
            pltpu.VMEM((CHG, 256), jnp.int32),
            pltpu.VMEM((CHG, 256), jnp.int32),
            pltpu.SemaphoreType.DMA,
            pltpu.SemaphoreType.DMA,
        ],
    )
    def run(v_hbm, src_hbm, vs_hbm, idx0, idx1, r0, r1, sem1, sem2):
        wid = lax.axis_index("s") * 2 + lax.axis_index("c")
        base0 = wid * per

        def half(base, n, idxb, rb, sem):
            ci = pltpu.async_copy(src_hbm.at[pl.ds(base, n)],
                                  idxb.at[pl.ds(0, n)], sem)
            ci.wait()
            cg = pltpu.async_copy(v_hbm.at[idxb.at[pl.ds(0, n)]],
                                  rb.at[pl.ds(0, n)], sem)
            return cg

        def step(i, carry):
            b = base0 + i * 2 * CHG
            c0 = half(b, CHG, idx0, r0, sem1)
            c1 = half(b + CHG, CHG, idx1, r1, sem2)
            c0.wait()
            w0 = pltpu.async_copy(r0, vs_hbm.at[pl.ds(b, CHG)], sem1)
            c1.wait()
            w1 = pltpu.async_copy(r1, vs_hbm.at[pl.ds(b + CHG, CHG)], sem2)
            w0.wait()
            w1.wait()
            return carry

        lax.fori_loop(0, npair, step, 0)
        b = base0 + npair * 2 * CHG
        c0 = half(b, CHG, idx0, r0, sem1)
        c1 = half(b + CHG, TL, idx1, r1, sem2)
        c0.wait()
        pltpu.async_copy(r0, vs_hbm.at[pl.ds(b, CHG)], sem1).wait()
        c1.wait()
        pltpu.async_copy(r1.at[pl.ds(0, TL)],
                         vs_hbm.at[pl.ds(b + CHG, TL)], sem2).wait()

    return run(v, src)


NP = 10240


def _scatter_den(ex, dst):
    per = (E // 2) // 16
    CHS = 128
    G = 2
    NR = NP // 16
    NF = 10
    FR = NR // NF

    @functools.partial(
        pl.kernel,
        out_type=jax.ShapeDtypeStruct((2, NP, 128), _f32),
        mesh=_mesh(),
        scratch_types=(
            [pltpu.VMEM_SHARED((NP, 128), _f32)]
            + [pltpu.VMEM((CHS,), jnp.int32)] * G
            + [pltpu.VMEM((CHS, 128), _f32)] * G
            + [pltpu.VMEM((FR, 128), _f32)]
            + [pltpu.SemaphoreType.DMA]
        ),
    )
    def run(ex_hbm, dst_hbm, den_hbm, acc,
            i0, i1, m0, m1,
            fbuf, semI):
        semM = semI
        c = lax.axis_index("c")
        sid = lax.axis_index("s")
        idxb = [i0, i1]
        mbuf = [m0, m1]

        def zrow(j, carry):
            for t in range(8):
                fbuf[j, pl.ds(t * 16, 16)] = jnp.zeros((16,), _f32)
            return carry

        lax.fori_loop(0, FR, zrow, 0)
        for f in range(NF):
            pltpu.sync_copy(fbuf, acc.at[pl.ds(sid * NR + f * FR, FR)])
        plsc.subcore_barrier()

        base0 = c * (E // 2) + sid * per

        def burst(gb, k):
            descs = []
            for j in range(k):
                b = gb + j * CHS
                descs.append(pltpu.async_copy(
                    dst_hbm.at[pl.ds(b, CHS)], idxb[j], semI))
                descs.append(pltpu.async_copy(
                    ex_hbm.at[pl.ds(b, CHS)], mbuf[j], semM))
            for d in descs:
                d.wait()
            for j in range(k):
                pltpu.sync_copy(mbuf[j], acc.at[idxb[j]], add=True)

        nch = per // CHS
        ngrp = nch // G
        rem = nch - ngrp * G
        tl = per - nch * CHS

        def group(i, carry):
            burst(base0 + i * G * CHS, G)
            return carry

        lax.fori_loop(0, ngrp, group, 0)
        burst(base0 + ngrp * G * CHS, rem)
        tb = base0 + nch * CHS
        pltpu.sync_copy(dst_hbm.at[pl.ds(tb, tl)], i0.at[pl.ds(0, tl)])
        pltpu.sync_copy(ex_hbm.at[pl.ds(tb, tl)], m0.at[pl.ds(0, tl)])
        pltpu.sync_copy(m0.at[pl.ds(0, tl)],
                        acc.at[i0.at[pl.ds(0, tl)]], add=True)

        plsc.subcore_barrier()
        for f in range(NF):
            pltpu.sync_copy(acc.at[pl.ds(sid * NR + f * FR, FR)], fbuf)
            pltpu.sync_copy(fbuf, den_hbm.at[c, pl.ds(sid * NR + f * FR, FR)])

    return run(ex, dst)


def _scatter_msg(ma, mb, mc, dst):
    CHS = 128
    G = 2
    per_full = E // 16
    per_half = (E // 2) // 16
    NR = NP // 16
    NF = 10
    FR = NR // NF

    @functools.partial(
        pl.kernel,
        out_type=[jax.ShapeDtypeStruct((2, NP, 128), _f32)] * 2,
        mesh=_mesh(),
        scratch_types=(
            [pltpu.VMEM_SHARED((NP, 128), _f32)]
            + [pltpu.VMEM((CHS,), jnp.int32)] * G
            + [pltpu.VMEM((CHS, 128), _f32)] * G
            + [pltpu.VMEM((FR, 128), _f32)]
            + [pltpu.SemaphoreType.DMA]
        ),
    )
    def run(ma_hbm, mb_hbm, mc_hbm, dst_hbm, o1_hbm, o2_hbm, acc,
            i0, i1, m0, m1,
            fbuf, semI):
        semM = semI
        c = lax.axis_index("c")
        sid = lax.axis_index("s")
        idxb = [i0, i1]
        mbuf = [m0, m1]

        def zrow(j, carry):
            for t in range(8):
                fbuf[j, pl.ds(t * 16, 16)] = jnp.zeros((16,), _f32)
            return carry

        def zero_acc():
            lax.fori_loop(0, FR, zrow, 0)
            for f in range(NF):
                pltpu.sync_copy(fbuf, acc.at[pl.ds(sid * NR + f * FR, FR)])

        def accumulate(m_hbm, base0, n_edges):
            def burst(gb, k):
                descs = []
                for j in range(k):
                    b = gb + j * CHS
                    descs.append(pltpu.async_copy(
                        dst_hbm.at[pl.ds(b, CHS)], idxb[j], semI))
                    descs.append(pltpu.async_copy(
                        m_hbm.at[pl.ds(b, CHS)], mbuf[j], semM))
                for d in descs:
                    d.wait()
                for j in range(k):
                    pltpu.sync_copy(mbuf[j], acc.at[idxb[j]], add=True)

            nch = n_edges // CHS
            ngrp = nch // G
            rem = nch - ngrp * G
            tl = n_edges - nch * CHS

            def group(i, carry):
                burst(base0 + i * G * CHS, G)
                return carry

            lax.fori_loop(0, ngrp, group, 0)
            if rem:
                burst(base0 + ngrp * G * CHS, rem)
            tb = base0 + nch * CHS
            pltpu.sync_copy(dst_hbm.at[pl.ds(tb, tl)], i0.at[pl.ds(0, tl)])
            pltpu.sync_copy(m_hbm.at[pl.ds(tb, tl)], m0.at[pl.ds(0, tl)])
            pltpu.sync_copy(m0.at[pl.ds(0, tl)],
                            acc.at[i0.at[pl.ds(0, tl)]], add=True)

        def flush(o_hbm):
            for f in range(NF):
                pltpu.sync_copy(acc.at[pl.ds(sid * NR + f * FR, FR)], fbuf)
                pltpu.sync_copy(fbuf, o_hbm.at[c, pl.ds(sid * NR + f * FR, FR)])

        zero_acc()
        plsc.subcore_barrier()

        @pl.when(c == 0)
        def _():
            accumulate(ma_hbm, sid * per_full, per_full)

        @pl.when(c == 1)
        def _():
            accumulate(mb_hbm, sid * per_full, per_full)

        plsc.subcore_barrier()
        flush(o1_hbm)
        plsc.subcore_barrier()

        zero_acc()
        plsc.subcore_barrier()
        accumulate(mc_hbm, c * (E // 2) + sid * per_half, per_half)
        plsc.subcore_barrier()
        flush(o2_hbm)

    return run(ma, mb, mc, dst)


def _gln_scales(st_row, count, w, b):
    m = st_row[0] / count
    var = st_row[1] / count - m * m
    sd = jnp.sqrt(jnp.maximum(var, 0.0))
    inv = 1.0 / (sd + EPS)
    return w * inv, b - m * w * inv


def kernel(x_graph, x_visual, x_prior, edge_index, edge_attr, batch, params):
    p = params
    src = edge_index[0]
    dst = edge_index[1]

    z_node, st_n = _encode_nodes(x_visual, x_graph, x_prior, p)
    ze, st_e = _encode_edges(edge_attr, p)

    cnt_n = float(N * 128)
    wv, bv = _gln_scales(st_n[0, 0:2], cnt_n, p['lnv_w'], p['lnv_b'])
    wg, bg = _gln_scales(st_n[0, 2:4], cnt_n, p['lng_w'], p['lng_b'])
    wp_, bp_ = _gln_scales(st_n[0, 4:6], cnt_n, p['lnp_w'], p['lnp_b'])
    wcat = jnp.concatenate([wv, wg, wp_]).reshape(1, D)
    bcat = jnp.concatenate([bv, bg, bp_]).reshape(1, D)
    h0 = _affine(z_node, wcat, bcat)

    we, be = _gln_scales(st_e[0, 0:2], float(E * 128), p['lne_w'], p['lne_b'])
    W1p = we[:, None] * p['Wedge1']
    b1p = be @ p['Wedge1']
    W2p = we[:, None] * p['Wedge2']
    b2p = be @ p['Wedge2']
    e1, e2 = _edgeproj(ze, W1p, b1p, W2p, b2p)

    hsel128 = (jnp.arange(D)[:, None] // C == jnp.arange(128)[None, :]).astype(_f32)
    hd = hsel128.T
    oh = (batch[:, None] == jnp.arange(128)[None, :]).astype(_f32)

    h = h0
    for s, e_l in (('1', e1), ('2', e2)):
        q, k, v = _qkv(h, p, s)
        def _pack(x):
            return lax.bitcast_convert_type(x.reshape(N, 256, 2), jnp.int32)

        def _unpack(x):
            return lax.bitcast_convert_type(x, jnp.bfloat16).reshape(E, 512)

        qd, ks = _gather_qk(_pack(q), _pack(k), dst, src)
        vs = _gather_v(_pack(v), src)
        ex, ma, mb, mc = _edge_attn(_unpack(qd), _unpack(ks),
                                    _unpack(vs), e_l, hsel128, hd)
        den_parts = _scatter_den(ex, dst)
        msg_full, msg_parts = _scatter_msg(ma, mb, mc, dst)
        t, st_g = _combine(h, msg_full, msg_parts, den_parts, hd,
                           p['Wskip' + s], p['bskip' + s], oh)
        cnt = jnp.maximum(st_g[:, 2], 1.0)
        mean = st_g[:, 0] / cnt
        var = st_g[:, 1] / cnt - mean * mean
        r = 1.0 / (jnp.sqrt(jnp.maximum(var, 0.0)) + EPS)
        mv = jnp.stack([mean, r], axis=1)
        h = _bnorm(t, oh, mv, p['ln' + s + '_w'], p['ln' + s + '_b'])

    Wc2p = jnp.pad(p['Wc2'], ((0, 0), (0, 128 - 49)))
    bc2p = jnp.pad(p['bc2'], (0, 128 - 49))
    logits = _final(h, p['Wc1'], p['bc1'], Wc2p, bc2p)
    return logits[:, :49]


# speedup vs baseline: 3.9081x; 3.9081x over previous
import functools

import jax
import jax.numpy as jnp
import numpy as np
from jax import lax
from jax.experimental import pallas as pl
from jax.experimental.pallas import tpu as pltpu

N = 10000
E = 160000
NG = 32
EPS = 1e-5
H, C = 8, 48
D = 384
BN = 1000
BE = 1000
RSQRT_C = 1.0 / np.sqrt(48.0)

_f32 = jnp.float32


def _cdiv(a, b):
    return (a + b - 1) // b


def _enc_node_body(xv_ref, xg_ref, xp_ref, Wv_ref, bv_ref, Wg_ref, bg_ref,
                   Wp_ref, bp_ref, z_ref, st_ref, acc):
    zv = jnp.maximum(jnp.dot(xv_ref[...], Wv_ref[...],
                             preferred_element_type=_f32) + bv_ref[...], 0.0)
    zg = jnp.maximum(jnp.dot(xg_ref[...], Wg_ref[...],
                             preferred_element_type=_f32) + bg_ref[...], 0.0)
    zp = jnp.maximum(jnp.dot(xp_ref[...], Wp_ref[...],
                             preferred_element_type=_f32) + bp_ref[...], 0.0)
    z_ref[...] = jnp.concatenate([zv, zg, zp], axis=1)
    vals = jnp.concatenate(
        [zv.sum(1, keepdims=True), (zv * zv).sum(1, keepdims=True),
         zg.sum(1, keepdims=True), (zg * zg).sum(1, keepdims=True),
         zp.sum(1, keepdims=True), (zp * zp).sum(1, keepdims=True)], axis=1)
    ones = jnp.full((8, vals.shape[0]), 1.0, _f32)
    part = jnp.dot(ones, vals, preferred_element_type=_f32)

    @pl.when(pl.program_id(0) == 0)
    def _():
        acc[...] = jnp.zeros_like(acc)

    acc[...] += part

    @pl.when(pl.program_id(0) == pl.num_programs(0) - 1)
    def _():
        st_ref[...] = acc[...]


def _encode_nodes(xv, xg, xp, p):
    grid = N // BN
    return pl.pallas_call(
        _enc_node_body,
        grid=(grid,),
        in_specs=[
            pl.BlockSpec((BN, 1024), lambda i: (i, 0)),
            pl.BlockSpec((BN, 6), lambda i: (i, 0)),
            pl.BlockSpec((BN, 50), lambda i: (i, 0)),
            pl.BlockSpec((1024, 128), lambda i: (0, 0)),
            pl.BlockSpec((1, 128), lambda i: (0, 0)),
            pl.BlockSpec((6, 128), lambda i: (0, 0)),
            pl.BlockSpec((1, 128), lambda i: (0, 0)),
            pl.BlockSpec((50, 128), lambda i: (0, 0)),
            pl.BlockSpec((1, 128), lambda i: (0, 0)),
        ],
        out_specs=[
            pl.BlockSpec((BN, D), lambda i: (i, 0)),
            pl.BlockSpec((8, 6), lambda i: (0, 0)),
        ],
        out_shape=[
            jax.ShapeDtypeStruct((N, D), _f32),
            jax.ShapeDtypeStruct((8, 6), _f32),
        ],
        scratch_shapes=[pltpu.VMEM((8, 6), _f32)],
    )(xv, xg, xp, p['Wvis'], p['bvis'].reshape(1, 128),
      p['Wg'], p['bg'].reshape(1, 128), p['Wp'], p['bp'].reshape(1, 128))


def _enc_edge_body(xe_ref, W_ref, b_ref, z_ref, st_ref, acc):
    z = jnp.maximum(jnp.dot(xe_ref[...], W_ref[...],
                            preferred_element_type=_f32) + b_ref[...], 0.0)
    z_ref[...] = z
    vals = jnp.concatenate(
        [z.sum(1, keepdims=True), (z * z).sum(1, keepdims=True)], axis=1)
    ones = jnp.full((8, vals.shape[0]), 1.0, _f32)
    part = jnp.dot(ones, vals, preferred_element_type=_f32)

    @pl.when(pl.program_id(0) == 0)
    def _():
        acc[...] = jnp.zeros_like(acc)

    acc[...] += part

    @pl.when(pl.program_id(0) == pl.num_programs(0) - 1)
    def _():
        st_ref[...] = acc[...]


def _encode_edges(xe, p):
    grid = E // BE
    return pl.pallas_call(
        _enc_edge_body,
        grid=(grid,),
        in_specs=[
            pl.BlockSpec((BE, 3), lambda i: (i, 0)),
            pl.BlockSpec((3, 128), lambda i: (0, 0)),
            pl.BlockSpec((1, 128), lambda i: (0, 0)),
        ],
        out_specs=[
            pl.BlockSpec((BE, 128), lambda i: (i, 0)),
            pl.BlockSpec((8, 2), lambda i: (0, 0)),
        ],
        out_shape=[
            jax.ShapeDtypeStruct((E, 128), _f32),
            jax.ShapeDtypeStruct((8, 2), _f32),
        ],
        scratch_shapes=[pltpu.VMEM((8, 2), _f32)],
    )(xe, p['Wee'], p['bee'].reshape(1, 128))


def _affine_body(z_ref, w_ref, b_ref, h_ref):
    h_ref[...] = z_ref[...] * w_ref[...] + b_ref[...]


def _affine(z, w_row, b_row):
    n, d = z.shape
    grid = n // BN
    return pl.pallas_call(
        _affine_body,
        grid=(grid,),
        in_specs=[
            pl.BlockSpec((BN, d), lambda i: (i, 0)),
            pl.BlockSpec((1, d), lambda i: (0, 0)),
            pl.BlockSpec((1, d), lambda i: (0, 0)),
        ],
        out_specs=pl.BlockSpec((BN, d), lambda i: (i, 0)),
        out_shape=jax.ShapeDtypeStruct((n, d), _f32),
    )(z, w_row, b_row)


def _qkv_body(h_ref, Wq_ref, bq_ref, Wk_ref, bk_ref, Wv_ref, bv_ref,
              q_ref, k_ref, v_ref):
    h = h_ref[...]
    q_ref[...] = jnp.dot(h, Wq_ref[...], preferred_element_type=_f32) + bq_ref[...]
    k_ref[...] = jnp.dot(h, Wk_ref[...], preferred_element_type=_f32) + bk_ref[...]
    v_ref[...] = jnp.dot(h, Wv_ref[...], preferred_element_type=_f32) + bv_ref[...]


def _qkv(h, p, s):
    grid = N // BN
    w = pl.BlockSpec((D, D), lambda i: (0, 0))
    b = pl.BlockSpec((1, D), lambda i: (0, 0))
    r = pl.BlockSpec((BN, D), lambda i: (i, 0))
    return pl.pallas_call(
        _qkv_body,
        grid=(grid,),
        in_specs=[r, w, b, w, b, w, b],
        out_specs=[r, r, r],
        out_shape=[jax.ShapeDtypeStruct((N, D), _f32)] * 3,
    )(h, p['Wq' + s], p['bq' + s].reshape(1, D),
      p['Wk' + s], p['bk' + s].reshape(1, D),
      p['Wval' + s], p['bval' + s].reshape(1, D))


def _edgeproj_body(z_ref, W1_ref, b1_ref, W2_ref, b2_ref, e1_ref, e2_ref):
    z = z_ref[...]
    e1_ref[...] = (jnp.dot(z, W1_ref[...], preferred_element_type=_f32)
                   + b1_ref[...]).astype(jnp.bfloat16)
    e2_ref[...] = (jnp.dot(z, W2_ref[...], preferred_element_type=_f32)
                   + b2_ref[...]).astype(jnp.bfloat16)


def _edgeproj(ze, W1, b1, W2, b2):
    grid = E // BE
    return pl.pallas_call(
        _edgeproj_body,
        grid=(grid,),
        in_specs=[
            pl.BlockSpec((BE, 128), lambda i: (i, 0)),
            pl.BlockSpec((128, D), lambda i: (0, 0)),
            pl.BlockSpec((1, D), lambda i: (0, 0)),
            pl.BlockSpec((128, D), lambda i: (0, 0)),
            pl.BlockSpec((1, D), lambda i: (0, 0)),
        ],
        out_specs=[pl.BlockSpec((BE, D), lambda i: (i, 0))] * 2,
        out_shape=[jax.ShapeDtypeStruct((E, D), jnp.bfloat16)] * 2,
    )(ze, W1, b1.reshape(1, D), W2, b2.reshape(1, D))


def _edge_body(qd_ref, ks_ref, vs_ref, e_ref, hsel_ref, hd_ref,
               ex_ref, ma_ref, mb_ref, mc_ref):
    e = e_ref[...].astype(_f32)
    t = qd_ref[...] * (ks_ref[...] + e)
    a = jnp.dot(t, hsel_ref[...], preferred_element_type=_f32) * RSQRT_C
    mask = (lax.broadcasted_iota(jnp.int32, a.shape, 1) < H).astype(_f32)
    ex = jnp.exp(a) * mask
    ex_ref[...] = ex
    a384 = jnp.dot(ex, hd_ref[...], preferred_element_type=_f32)
    m = (vs_ref[...] + e) * a384
    ma_ref[...] = m[:, :128]
    mb_ref[...] = m[:, 128:256]
    mc_ref[...] = m[:, 256:]


def _edge_attn(qd, ks, vs, e, hsel128, hd):
    grid = E // BE
    r = pl.BlockSpec((BE, D), lambda i: (i, 0))
    r128 = pl.BlockSpec((BE, 128), lambda i: (i, 0))
    return pl.pallas_call(
        _edge_body,
        grid=(grid,),
        in_specs=[r, r, r, r,
                  pl.BlockSpec((D, 128), lambda i: (0, 0)),
                  pl.BlockSpec((128, D), lambda i: (0, 0))],
        out_specs=[r128] * 4,
        out_shape=[jax.ShapeDtypeStruct((E, 128), _f32)] * 4,
    )(qd, ks, vs, e, hsel128, hd)


def _combine_body(h_ref, a_ref, b_ref, c0_ref, c1_ref, d0_ref, d1_ref,
                  hd_ref, Ws_ref, bs_ref, oh_ref, t_ref, st_ref, acc):
    h = h_ref[...]
    den = d0_ref[...] + d1_ref[...]
    den_exp = jnp.dot(den, hd_ref[...], preferred_element_type=_f32)
    o = jnp.concatenate([a_ref[...], b_ref[...], c0_ref[...] + c1_ref[...]],
                        axis=1)
    o = o / (den_exp + 1e-16)
    o = o + jnp.dot(h, Ws_ref[...], preferred_element_type=_f32) + bs_ref[...]
    t = h + jnp.where(o > 0, o, jnp.exp(jnp.minimum(o, 0.0)) - 1.0)
    t_ref[...] = t
    rs = t.sum(1, keepdims=True)
    rq = (t * t).sum(1, keepdims=True)
    vals = jnp.concatenate([rs, rq, jnp.full_like(rs, float(D))], axis=1)
    part = lax.dot_general(oh_ref[...], vals, (((0,), (0,)), ((), ())),
                           preferred_element_type=_f32)

    @pl.when(pl.program_id(0) == 0)
    def _():
        acc[...] = jnp.zeros_like(acc)

    acc[...] += part

    @pl.when(pl.program_id(0) == pl.num_programs(0) - 1)
    def _():
        st_ref[...] = acc[...]


def _combine(h, msg_full, msg_parts, den_parts, hd, Ws, bs, oh):
    grid = N // BN
    r128 = pl.BlockSpec((BN, 128), lambda i: (i, 0))
    return pl.pallas_call(
        _combine_body,
        grid=(grid,),
        in_specs=[
            pl.BlockSpec((BN, D), lambda i: (i, 0)),
            r128, r128, r128, r128, r128, r128,
            pl.BlockSpec((128, D), lambda i: (0, 0)),
            pl.BlockSpec((D, D), lambda i: (0, 0)),
            pl.BlockSpec((1, D), lambda i: (0, 0)),
            r128,
        ],
        out_specs=[
            pl.BlockSpec((BN, D), lambda i: (i, 0)),
            pl.BlockSpec((128, 3), lambda i: (0, 0)),
        ],
        out_shape=[
            jax.ShapeDtypeStruct((N, D), _f32),
            jax.ShapeDtypeStruct((128, 3), _f32),
        ],
        scratch_shapes=[pltpu.VMEM((128, 3), _f32)],
    )(h, msg_full[0, :N], msg_full[1, :N], msg_parts[0, :N], msg_parts[1, :N],
      den_parts[0, :N], den_parts[1, :N], hd, Ws, bs.reshape(1, D), oh)


def _bnorm_body(t_ref, oh_ref, mv_ref, w_ref, b_ref, h_ref):
    mg = jnp.dot(oh_ref[...], mv_ref[...], preferred_element_type=_f32)
    m = mg[:, 0:1]
    r = mg[:, 1:2]
    h_ref[...] = (t_ref[...] - m) * r * w_ref[...] + b_ref[...]


def _bnorm(t, oh, mv, w, b):
    grid = N // BN
    return pl.pallas_call(
        _bnorm_body,
        grid=(grid,),
        in_specs=[
            pl.BlockSpec((BN, D), lambda i: (i, 0)),
            pl.BlockSpec((BN, 128), lambda i: (i, 0)),
            pl.BlockSpec((128, 2), lambda i: (0, 0)),
            pl.BlockSpec((1, D), lambda i: (0, 0)),
            pl.BlockSpec((1, D), lambda i: (0, 0)),
        ],
        out_specs=pl.BlockSpec((BN, D), lambda i: (i, 0)),
        out_shape=jax.ShapeDtypeStruct((N, D), _f32),
    )(t, oh, mv, w.reshape(1, D), b.reshape(1, D))


def _final_body(h_ref, W1_ref, b1_ref, W2_ref, b2_ref, o_ref):
    l1 = jnp.maximum(jnp.dot(h_ref[...], W1_ref[...],
                             preferred_element_type=_f32) + b1_ref[...], 0.0)
    o_ref[...] = jnp.dot(l1, W2_ref[...], preferred_element_type=_f32) + b2_ref[...]


def _final(h2, W1, b1, W2p, b2p):
    grid = N // BN
    return pl.pallas_call(
        _final_body,
        grid=(grid,),
        in_specs=[
            pl.BlockSpec((BN, D), lambda i: (i, 0)),
            pl.BlockSpec((D, 128), lambda i: (0, 0)),
            pl.BlockSpec((1, 128), lambda i: (0, 0)),
            pl.BlockSpec((128, 128), lambda i: (0, 0)),
            pl.BlockSpec((1, 128), lambda i: (0, 0)),
        ],
        out_specs=pl.BlockSpec((BN, 128), lambda i: (i, 0)),
        out_shape=jax.ShapeDtypeStruct((N, 128), _f32),
    )(h2, W1, b1.reshape(1, 128), W2p, b2p.reshape(1, 128))


from jax.experimental.pallas import tpu_sc as plsc

NW = 32
CH = 40


def _mesh():
    return plsc.VectorSubcoreMesh(core_axis_name="c", subcore_axis_name="s")


def _gather_qk(q, k, dst, src):
    per = E // NW
    CHG = 128
    nch = per // CHG
    TL = per - nch * CHG
    @functools.partial(
        pl.kernel,
        out_type=[jax.ShapeDtypeStruct((E, D), _f32)] * 2,
        mesh=_mesh(),
        scratch_types=[
            pltpu.VMEM((CHG,), jnp.int32),
            pltpu.VMEM((CHG,), jnp.int32),
            pltpu.VMEM((CHG, D), _f32),
            pltpu.VMEM((CHG, D), _f32),
            pltpu.SemaphoreType.DMA,
            pltpu.SemaphoreType.DMA,
        ],
    )
    def run(q_hbm, k_hbm, dst_hbm, src_hbm, qd_hbm, ks_hbm,
            idxd, idxs, qrows, krows, sem1, sem2):
        wid = lax.axis_index("s") * 2 + lax.axis_index("c")
        base0 = wid * per

        def chunk(base, n):
            ci = pltpu.async_copy(dst_hbm.at[pl.ds(base, n)],
                                  idxd.at[pl.ds(0, n)], sem1)
            cj = pltpu.async_copy(src_hbm.at[pl.ds(base, n)],
                                  idxs.at[pl.ds(0, n)], sem2)
            ci.wait()
            cj.wait()
            cq = pltpu.async_copy(q_hbm.at[idxd.at[pl.ds(0, n)]],
                                  qrows.at[pl.ds(0, n)], sem1)
            ck = pltpu.async_copy(k_hbm.at[idxs.at[pl.ds(0, n)]],
                                  krows.at[pl.ds(0, n)], sem2)
            cq.wait()
            ck.wait()
            wq = pltpu.async_copy(qrows.at[pl.ds(0, n)],
                                  qd_hbm.at[pl.ds(base, n)], sem1)
            wk = pltpu.async_copy(krows.at[pl.ds(0, n)],
                                  ks_hbm.at[pl.ds(base, n)], sem2)
            wq.wait()
            wk.wait()

        def step(i, carry):
            chunk(base0 + i * CHG, CHG)
            return carry

        lax.fori_loop(0, nch, step, 0)
        chunk(base0 + nch * CHG, TL)

    return run(q, k, dst, src)


def _gather_v(v, src):
    per = E // NW
    CHG = 128
    nch = per // CHG
    TL = per - nch * CHG
    npair = nch // 2

    @functools.partial(
        pl.kernel,
        out_type=jax.ShapeDtypeStruct((E, D), _f32),
        mesh=_mesh(),
        scratch_types=[
            pltpu.VMEM((CHG,), jnp.int32),
            pltpu.VMEM((CHG,), jnp.int32),
            pltpu.VMEM((CHG, D), _f32),
            pltpu.VMEM((CHG, D), _f32),
            pltpu.SemaphoreType.DMA,
            pltpu.SemaphoreType.DMA,
        ],
    )
    def run(v_hbm, src_hbm, vs_hbm, idx0, idx1, r0, r1, sem1, sem2):
        wid = lax.axis_index("s") * 2 + lax.axis_index("c")
        base0 = wid * per

        def half(base, n, idxb, rb, sem):
            ci = pltpu.async_copy(src_hbm.at[pl.ds(base, n)],
                                  idxb.at[pl.ds(0, n)], sem)
            ci.wait()
            cg = pltpu.async_copy(v_hbm.at[idxb.at[pl.ds(0, n)]],
                                  rb.at[pl.ds(0, n)], sem)
            return cg

        def step(i, carry):
            b = base0 + i * 2 * CHG
            c0 = half(b, CHG, idx0, r0, sem1)
            c1 = half(b + CHG, CHG, idx1, r1, sem2)
            c0.wait()
            w0 = pltpu.async_copy(r0, vs_hbm.at[pl.ds(b, CHG)], sem1)
            c1.wait()
            w1 = pltpu.async_copy(r1, vs_hbm.at[pl.ds(b + CHG, CHG)], sem2)
            w0.wait()
            w1.wait()
            return carry

        lax.fori_loop(0, npair, step, 0)
        b = base0 + npair * 2 * CHG
        c0 = half(b, CHG, idx0, r0, sem1)
        c1 = half(b + CHG, TL, idx1, r1, sem2)
        c0.wait()
        pltpu.async_copy(r0, vs_hbm.at[pl.ds(b, CHG)], sem1).wait()
        c1.wait()
        pltpu.async_copy(r1.at[pl.ds(0, TL)],
                         vs_hbm.at[pl.ds(b + CHG, TL)], sem2).wait()

    return run(v, src)


NP = 10240


def _scatter_den(ex, dst):
    per = (E // 2) // 16
    CHS = 128
    G = 2
    NR = NP // 16
    NF = 10
    FR = NR // NF

    @functools.partial(
        pl.kernel,
        out_type=jax.ShapeDtypeStruct((2, NP, 128), _f32),
        mesh=_mesh(),
        scratch_types=(
            [pltpu.VMEM_SHARED((NP, 128), _f32)]
            + [pltpu.VMEM((CHS,), jnp.int32)] * G
            + [pltpu.VMEM((CHS, 128), _f32)] * G
            + [pltpu.VMEM((FR, 128), _f32)]
            + [pltpu.SemaphoreType.DMA]
        ),
    )
    def run(ex_hbm, dst_hbm, den_hbm, acc,
            i0, i1, m0, m1,
            fbuf, semI):
        semM = semI
        c = lax.axis_index("c")
        sid = lax.axis_index("s")
        idxb = [i0, i1]
        mbuf = [m0, m1]

        def zrow(j, carry):
            for t in range(8):
                fbuf[j, pl.ds(t * 16, 16)] = jnp.zeros((16,), _f32)
            return carry

        lax.fori_loop(0, FR, zrow, 0)
        for f in range(NF):
            pltpu.sync_copy(fbuf, acc.at[pl.ds(sid * NR + f * FR, FR)])
        plsc.subcore_barrier()

        base0 = c * (E // 2) + sid * per

        def burst(gb, k):
            descs = []
            for j in range(k):
                b = gb + j * CHS
                descs.append(pltpu.async_copy(
                    dst_hbm.at[pl.ds(b, CHS)], idxb[j], semI))
                descs.append(pltpu.async_copy(
                    ex_hbm.at[pl.ds(b, CHS)], mbuf[j], semM))
            for d in descs:
                d.wait()
            for j in range(k):
                pltpu.sync_copy(mbuf[j], acc.at[idxb[j]], add=True)

        nch = per // CHS
        ngrp = nch // G
        rem = nch - ngrp * G
        tl = per - nch * CHS

        def group(i, carry):
            burst(base0 + i * G * CHS, G)
            return carry

        lax.fori_loop(0, ngrp, group, 0)
        burst(base0 + ngrp * G * CHS, rem)
        tb = base0 + nch * CHS
        pltpu.sync_copy(dst_hbm.at[pl.ds(tb, tl)], i0.at[pl.ds(0, tl)])
        pltpu.sync_copy(ex_hbm.at[pl.ds(tb, tl)], m0.at[pl.ds(0, tl)])
        pltpu.sync_copy(m0.at[pl.ds(0, tl)],
                        acc.at[i0.at[pl.ds(0, tl)]], add=True)

        plsc.subcore_barrier()
        for f in range(NF):
            pltpu.sync_copy(acc.at[pl.ds(sid * NR + f * FR, FR)], fbuf)
            pltpu.sync_copy(fbuf, den_hbm.at[c, pl.ds(sid * NR + f * FR, FR)])

    return run(ex, dst)


def _scatter_msg(ma, mb, mc, dst):
    CHS = 128
    G = 2
    per_full = E // 16
    per_half = (E // 2) // 16
    NR = NP // 16
    NF = 10
    FR = NR // NF

    @functools.partial(
        pl.kernel,
        out_type=[jax.ShapeDtypeStruct((2, NP, 128), _f32)] * 2,
        mesh=_mesh(),
        scratch_types=(
            [pltpu.VMEM_SHARED((NP, 128), _f32)]
            + [pltpu.VMEM((CHS,), jnp.int32)] * G
            + [pltpu.VMEM((CHS, 128), _f32)] * G
            + [pltpu.VMEM((FR, 128), _f32)]
            + [pltpu.SemaphoreType.DMA]
        ),
    )
    def run(ma_hbm, mb_hbm, mc_hbm, dst_hbm, o1_hbm, o2_hbm, acc,
            i0, i1, m0, m1,
            fbuf, semI):
        semM = semI
        c = lax.axis_index("c")
        sid = lax.axis_index("s")
        idxb = [i0, i1]
        mbuf = [m0, m1]

        def zrow(j, carry):
            for t in range(8):
                fbuf[j, pl.ds(t * 16, 16)] = jnp.zeros((16,), _f32)
            return carry

        def zero_acc():
            lax.fori_loop(0, FR, zrow, 0)
            for f in range(NF):
                pltpu.sync_copy(fbuf, acc.at[pl.ds(sid * NR + f * FR, FR)])

        def accumulate(m_hbm, base0, n_edges):
            def burst(gb, k):
                descs = []
                for j in range(k):
                    b = gb + j * CHS
                    descs.append(pltpu.async_copy(
                        dst_hbm.at[pl.ds(b, CHS)], idxb[j], semI))
                    descs.append(pltpu.async_copy(
                        m_hbm.at[pl.ds(b, CHS)], mbuf[j], semM))
                for d in descs:
                    d.wait()
                for j in range(k):
                    pltpu.sync_copy(mbuf[j], acc.at[idxb[j]], add=True)

            nch = n_edges // CHS
            ngrp = nch // G
            rem = nch - ngrp * G
            tl = n_edges - nch * CHS

            def group(i, carry):
                burst(base0 + i * G * CHS, G)
                return carry

            lax.fori_loop(0, ngrp, group, 0)
            if rem:
                burst(base0 + ngrp * G * CHS, rem)
            tb = base0 + nch * CHS
            pltpu.sync_copy(dst_hbm.at[pl.ds(tb, tl)], i0.at[pl.ds(0, tl)])
            pltpu.sync_copy(m_hbm.at[pl.ds(tb, tl)], m0.at[pl.ds(0, tl)])
            pltpu.sync_copy(m0.at[pl.ds(0, tl)],
                            acc.at[i0.at[pl.ds(0, tl)]], add=True)

        def flush(o_hbm):
            for f in range(NF):
                pltpu.sync_copy(acc.at[pl.ds(sid * NR + f * FR, FR)], fbuf)
                pltpu.sync_copy(fbuf, o_hbm.at[c, pl.ds(sid * NR + f * FR, FR)])

        zero_acc()
        plsc.subcore_barrier()

        @pl.when(c == 0)
        def _():
            accumulate(ma_hbm, sid * per_full, per_full)

        @pl.when(c == 1)
        def _():
            accumulate(mb_hbm, sid * per_full, per_full)

        plsc.subcore_barrier()
        flush(o1_hbm)
        plsc.subcore_barrier()

        zero_acc()
        plsc.subcore_barrier()
        accumulate(mc_hbm, c * (E // 2) + sid * per_half, per_half)
        plsc.subcore_barrier()
        flush(o2_hbm)

    return run(ma, mb, mc, dst)


def _gln_scales(st_row, count, w, b):
    m = st_row[0] / count
    var = st_row[1] / count - m * m
    sd = jnp.sqrt(jnp.maximum(var, 0.0))
    inv = 1.0 / (sd + EPS)
    return w * inv, b - m * w * inv


def kernel(x_graph, x_visual, x_prior, edge_index, edge_attr, batch, params):
    p = params
    src = edge_index[0]
    dst = edge_index[1]

    z_node, st_n = _encode_nodes(x_visual, x_graph, x_prior, p)
    ze, st_e = _encode_edges(edge_attr, p)

    cnt_n = float(N * 128)
    wv, bv = _gln_scales(st_n[0, 0:2], cnt_n, p['lnv_w'], p['lnv_b'])
    wg, bg = _gln_scales(st_n[0, 2:4], cnt_n, p['lng_w'], p['lng_b'])
    wp_, bp_ = _gln_scales(st_n[0, 4:6], cnt_n, p['lnp_w'], p['lnp_b'])
    wcat = jnp.concatenate([wv, wg, wp_]).reshape(1, D)
    bcat = jnp.concatenate([bv, bg, bp_]).reshape(1, D)
    h0 = _affine(z_node, wcat, bcat)

    we, be = _gln_scales(st_e[0, 0:2], float(E * 128), p['lne_w'], p['lne_b'])
    W1p = we[:, None] * p['Wedge1']
    b1p = be @ p['Wedge1']
    W2p = we[:, None] * p['Wedge2']
    b2p = be @ p['Wedge2']
    e1, e2 = _edgeproj(ze, W1p, b1p, W2p, b2p)

    hsel128 = (jnp.arange(D)[:, None] // C == jnp.arange(128)[None, :]).astype(_f32)
    hd = hsel128.T
    oh = (batch[:, None] == jnp.arange(128)[None, :]).astype(_f32)

    h = h0
    for s, e_l in (('1', e1), ('2', e2)):
        q, k, v = _qkv(h, p, s)
        qd, ks = _gather_qk(q, k, dst, src)
        vs = _gather_v(v, src)
        ex, ma, mb, mc = _edge_attn(qd, ks, vs, e_l, hsel128, hd)
        den_parts = _scatter_den(ex, dst)
        msg_full, msg_parts = _scatter_msg(ma, mb, mc, dst)
        t, st_g = _combine(h, msg_full, msg_parts, den_parts, hd,
                           p['Wskip' + s], p['bskip' + s], oh)
        cnt = jnp.maximum(st_g[:, 2], 1.0)
        mean = st_g[:, 0] / cnt
        var = st_g[:, 1] / cnt - mean * mean
        r = 1.0 / (jnp.sqrt(jnp.maximum(var, 0.0)) + EPS)
        mv = jnp.stack([mean, r], axis=1)
        h = _bnorm(t, oh, mv, p['ln' + s + '_w'], p['ln' + s + '_b'])

    Wc2p = jnp.pad(p['Wc2'], ((0, 0), (0, 128 - 49)))
    bc2p = jnp.pad(p['bc2'], (0, 128 - 49))
    logits = _final(h, p['Wc1'], p['bc1'], Wc2p, bc2p)
    return logits[:, :49]


# speedup vs baseline: 3.9127x; 1.0012x over previous
import functools

import jax
import jax.numpy as jnp
import numpy as np
from jax import lax
from jax.experimental import pallas as pl
from jax.experimental.pallas import tpu as pltpu

N = 10000
E = 160000
EPS = 1e-5
H, C = 8, 48
D = 384
BN = 1000
BE = 1000
RSQRT_C = 1.0 / np.sqrt(48.0)

_f32 = jnp.float32


def _enc_node_body(xv_ref, xg_ref, xp_ref, Wv_ref, bv_ref, Wg_ref, bg_ref,
                   Wp_ref, bp_ref, z_ref, st_ref, acc):
    zv = jnp.maximum(jnp.dot(xv_ref[...], Wv_ref[...],
                             preferred_element_type=_f32) + bv_ref[...], 0.0)
    zg = jnp.maximum(jnp.dot(xg_ref[...], Wg_ref[...],
                             preferred_element_type=_f32) + bg_ref[...], 0.0)
    zp = jnp.maximum(jnp.dot(xp_ref[...], Wp_ref[...],
                             preferred_element_type=_f32) + bp_ref[...], 0.0)
    z_ref[...] = jnp.concatenate([zv, zg, zp], axis=1)
    vals = jnp.concatenate(
        [zv.sum(1, keepdims=True), (zv * zv).sum(1, keepdims=True),
         zg.sum(1, keepdims=True), (zg * zg).sum(1, keepdims=True),
         zp.sum(1, keepdims=True), (zp * zp).sum(1, keepdims=True)], axis=1)
    ones = jnp.full((8, vals.shape[0]), 1.0, _f32)
    part = jnp.dot(ones, vals, preferred_element_type=_f32)

    @pl.when(pl.program_id(0) == 0)
    def _():
        acc[...] = jnp.zeros_like(acc)

    acc[...] += part

    @pl.when(pl.program_id(0) == pl.num_programs(0) - 1)
    def _():
        st_ref[...] = acc[...]


def _encode_nodes(xv, xg, xp, p):
    grid = N // BN
    return pl.pallas_call(
        _enc_node_body,
        grid=(grid,),
        in_specs=[
            pl.BlockSpec((BN, 1024), lambda i: (i, 0)),
            pl.BlockSpec((BN, 6), lambda i: (i, 0)),
            pl.BlockSpec((BN, 50), lambda i: (i, 0)),
            pl.BlockSpec((1024, 128), lambda i: (0, 0)),
            pl.BlockSpec((1, 128), lambda i: (0, 0)),
            pl.BlockSpec((6, 128), lambda i: (0, 0)),
            pl.BlockSpec((1, 128), lambda i: (0, 0)),
            pl.BlockSpec((50, 128), lambda i: (0, 0)),
            pl.BlockSpec((1, 128), lambda i: (0, 0)),
        ],
        out_specs=[
            pl.BlockSpec((BN, D), lambda i: (i, 0)),
            pl.BlockSpec((8, 6), lambda i: (0, 0)),
        ],
        out_shape=[
            jax.ShapeDtypeStruct((N, D), _f32),
            jax.ShapeDtypeStruct((8, 6), _f32),
        ],
        scratch_shapes=[pltpu.VMEM((8, 6), _f32)],
    )(xv, xg, xp, p['Wvis'], p['bvis'].reshape(1, 128),
      p['Wg'], p['bg'].reshape(1, 128), p['Wp'], p['bp'].reshape(1, 128))


def _enc_edge_body(xe_ref, W_ref, b_ref, z_ref, st_ref, acc):
    z = jnp.maximum(jnp.dot(xe_ref[...], W_ref[...],
                            preferred_element_type=_f32) + b_ref[...], 0.0)
    z_ref[...] = z
    vals = jnp.concatenate(
        [z.sum(1, keepdims=True), (z * z).sum(1, keepdims=True)], axis=1)
    ones = jnp.full((8, vals.shape[0]), 1.0, _f32)
    part = jnp.dot(ones, vals, preferred_element_type=_f32)

    @pl.when(pl.program_id(0) == 0)
    def _():
        acc[...] = jnp.zeros_like(acc)

    acc[...] += part

    @pl.when(pl.program_id(0) == pl.num_programs(0) - 1)
    def _():
        st_ref[...] = acc[...]


def _encode_edges(xe, p):
    grid = E // BE
    return pl.pallas_call(
        _enc_edge_body,
        grid=(grid,),
        in_specs=[
            pl.BlockSpec((BE, 3), lambda i: (i, 0)),
            pl.BlockSpec((3, 128), lambda i: (0, 0)),
            pl.BlockSpec((1, 128), lambda i: (0, 0)),
        ],
        out_specs=[
            pl.BlockSpec((BE, 128), lambda i: (i, 0)),
            pl.BlockSpec((8, 2), lambda i: (0, 0)),
        ],
        out_shape=[
            jax.ShapeDtypeStruct((E, 128), _f32),
            jax.ShapeDtypeStruct((8, 2), _f32),
        ],
        scratch_shapes=[pltpu.VMEM((8, 2), _f32)],
    )(xe, p['Wee'], p['bee'].reshape(1, 128))


def _affine_body(z_ref, w_ref, b_ref, h_ref):
    h_ref[...] = z_ref[...] * w_ref[...] + b_ref[...]


def _affine(z, w_row, b_row):
    n, d = z.shape
    grid = n // BN
    return pl.pallas_call(
        _affine_body,
        grid=(grid,),
        in_specs=[
            pl.BlockSpec((BN, d), lambda i: (i, 0)),
            pl.BlockSpec((1, d), lambda i: (0, 0)),
            pl.BlockSpec((1, d), lambda i: (0, 0)),
        ],
        out_specs=pl.BlockSpec((BN, d), lambda i: (i, 0)),
        out_shape=jax.ShapeDtypeStruct((n, d), _f32),
    )(z, w_row, b_row)


def _qkv_body(h_ref, Wq_ref, bq_ref, Wk_ref, bk_ref, Wv_ref, bv_ref,
              q_ref, k_ref, v_ref):
    h = h_ref[...]
    q_ref[...] = jnp.dot(h, Wq_ref[...], preferred_element_type=_f32) + bq_ref[...]
    k_ref[...] = jnp.dot(h, Wk_ref[...], preferred_element_type=_f32) + bk_ref[...]
    v_ref[...] = jnp.dot(h, Wv_ref[...], preferred_element_type=_f32) + bv_ref[...]


def _qkv(h, p, s):
    grid = N // BN
    w = pl.BlockSpec((D, D), lambda i: (0, 0))
    b = pl.BlockSpec((1, D), lambda i: (0, 0))
    r = pl.BlockSpec((BN, D), lambda i: (i, 0))
    return pl.pallas_call(
        _qkv_body,
        grid=(grid,),
        in_specs=[r, w, b, w, b, w, b],
        out_specs=[r, r, r],
        out_shape=[jax.ShapeDtypeStruct((N, D), _f32)] * 3,
    )(h, p['Wq' + s], p['bq' + s].reshape(1, D),
      p['Wk' + s], p['bk' + s].reshape(1, D),
      p['Wval' + s], p['bval' + s].reshape(1, D))


def _edgeproj_body(z_ref, W1_ref, b1_ref, W2_ref, b2_ref, e1_ref, e2_ref):
    z = z_ref[...]
    e1_ref[...] = (jnp.dot(z, W1_ref[...], preferred_element_type=_f32)
                   + b1_ref[...]).astype(jnp.bfloat16)
    e2_ref[...] = (jnp.dot(z, W2_ref[...], preferred_element_type=_f32)
                   + b2_ref[...]).astype(jnp.bfloat16)


def _edgeproj(ze, W1, b1, W2, b2):
    grid = E // BE
    return pl.pallas_call(
        _edgeproj_body,
        grid=(grid,),
        in_specs=[
            pl.BlockSpec((BE, 128), lambda i: (i, 0)),
            pl.BlockSpec((128, D), lambda i: (0, 0)),
            pl.BlockSpec((1, D), lambda i: (0, 0)),
            pl.BlockSpec((128, D), lambda i: (0, 0)),
            pl.BlockSpec((1, D), lambda i: (0, 0)),
        ],
        out_specs=[pl.BlockSpec((BE, D), lambda i: (i, 0))] * 2,
        out_shape=[jax.ShapeDtypeStruct((E, D), jnp.bfloat16)] * 2,
    )(ze, W1, b1.reshape(1, D), W2, b2.reshape(1, D))


def _edge_body(qd_ref, ks_ref, vs_ref, e_ref, hsel_ref, hd_ref,
               ex_ref, ma_ref, mb_ref, mc_ref):
    e = e_ref[...].astype(_f32)
    t = qd_ref[...] * (ks_ref[...] + e)
    a = jnp.dot(t, hsel_ref[...], preferred_element_type=_f32) * RSQRT_C
    mask = (lax.broadcasted_iota(jnp.int32, a.shape, 1) < H).astype(_f32)
    ex = jnp.exp(a) * mask
    ex_ref[...] = ex
    a384 = jnp.dot(ex, hd_ref[...], preferred_element_type=_f32)
    m = (vs_ref[...] + e) * a384
    ma_ref[...] = m[:, :128]
    mb_ref[...] = m[:, 128:256]
    mc_ref[...] = m[:, 256:]


def _edge_attn(qd, ks, vs, e, hsel128, hd):
    grid = E // BE
    r = pl.BlockSpec((BE, D), lambda i: (i, 0))
    r128 = pl.BlockSpec((BE, 128), lambda i: (i, 0))
    return pl.pallas_call(
        _edge_body,
        grid=(grid,),
        in_specs=[r, r, r, r,
                  pl.BlockSpec((D, 128), lambda i: (0, 0)),
                  pl.BlockSpec((128, D), lambda i: (0, 0))],
        out_specs=[r128] * 4,
        out_shape=[jax.ShapeDtypeStruct((E, 128), _f32)] * 4,
    )(qd, ks, vs, e, hsel128, hd)


def _combine_body(h_ref, a_ref, b_ref, c0_ref, c1_ref, d0_ref, d1_ref,
                  hd_ref, Ws_ref, bs_ref, oh_ref, t_ref, st_ref, acc):
    h = h_ref[...]
    den = d0_ref[...] + d1_ref[...]
    den_exp = jnp.dot(den, hd_ref[...], preferred_element_type=_f32)
    o = jnp.concatenate([a_ref[...], b_ref[...], c0_ref[...] + c1_ref[...]],
                        axis=1)
    o = o / (den_exp + 1e-16)
    o = o + jnp.dot(h, Ws_ref[...], preferred_element_type=_f32) + bs_ref[...]
    t = h + jnp.where(o > 0, o, jnp.exp(jnp.minimum(o, 0.0)) - 1.0)
    t_ref[...] = t
    rs = t.sum(1, keepdims=True)
    rq = (t * t).sum(1, keepdims=True)
    vals = jnp.concatenate([rs, rq, jnp.full_like(rs, float(D))], axis=1)
    part = lax.dot_general(oh_ref[...], vals, (((0,), (0,)), ((), ())),
                           preferred_element_type=_f32)

    @pl.when(pl.program_id(0) == 0)
    def _():
        acc[...] = jnp.zeros_like(acc)

    acc[...] += part

    @pl.when(pl.program_id(0) == pl.num_programs(0) - 1)
    def _():
        st_ref[...] = acc[...]


def _combine(h, msg_full, msg_parts, den_parts, hd, Ws, bs, oh):
    grid = N // BN
    r128 = pl.BlockSpec((BN, 128), lambda i: (i, 0))
    return pl.pallas_call(
        _combine_body,
        grid=(grid,),
        in_specs=[
            pl.BlockSpec((BN, D), lambda i: (i, 0)),
            r128, r128, r128, r128, r128, r128,
            pl.BlockSpec((128, D), lambda i: (0, 0)),
            pl.BlockSpec((D, D), lambda i: (0, 0)),
            pl.BlockSpec((1, D), lambda i: (0, 0)),
            r128,
        ],
        out_specs=[
            pl.BlockSpec((BN, D), lambda i: (i, 0)),
            pl.BlockSpec((128, 3), lambda i: (0, 0)),
        ],
        out_shape=[
            jax.ShapeDtypeStruct((N, D), _f32),
            jax.ShapeDtypeStruct((128, 3), _f32),
        ],
        scratch_shapes=[pltpu.VMEM((128, 3), _f32)],
    )(h, msg_full[0, :N], msg_full[1, :N], msg_parts[0, :N], msg_parts[1, :N],
      den_parts[0, :N], den_parts[1, :N], hd, Ws, bs.reshape(1, D), oh)


def _bnorm_body(t_ref, oh_ref, mv_ref, w_ref, b_ref, h_ref):
    mg = jnp.dot(oh_ref[...], mv_ref[...], preferred_element_type=_f32)
    m = mg[:, 0:1]
    r = mg[:, 1:2]
    h_ref[...] = (t_ref[...] - m) * r * w_ref[...] + b_ref[...]


def _bnorm(t, oh, mv, w, b):
    grid = N // BN
    return pl.pallas_call(
        _bnorm_body,
        grid=(grid,),
        in_specs=[
            pl.BlockSpec((BN, D), lambda i: (i, 0)),
            pl.BlockSpec((BN, 128), lambda i: (i, 0)),
            pl.BlockSpec((128, 2), lambda i: (0, 0)),
            pl.BlockSpec((1, D), lambda i: (0, 0)),
            pl.BlockSpec((1, D), lambda i: (0, 0)),
        ],
        out_specs=pl.BlockSpec((BN, D), lambda i: (i, 0)),
        out_shape=jax.ShapeDtypeStruct((N, D), _f32),
    )(t, oh, mv, w.reshape(1, D), b.reshape(1, D))


def _final_body(h_ref, W1_ref, b1_ref, W2_ref, b2_ref, o_ref):
    l1 = jnp.maximum(jnp.dot(h_ref[...], W1_ref[...],
                             preferred_element_type=_f32) + b1_ref[...], 0.0)
    o_ref[...] = jnp.dot(l1, W2_ref[...], preferred_element_type=_f32) + b2_ref[...]


def _final(h2, W1, b1, W2p, b2p):
    grid = N // BN
    return pl.pallas_call(
        _final_body,
        grid=(grid,),
        in_specs=[
            pl.BlockSpec((BN, D), lambda i: (i, 0)),
            pl.BlockSpec((D, 128), lambda i: (0, 0)),
            pl.BlockSpec((1, 128), lambda i: (0, 0)),
            pl.BlockSpec((128, 128), lambda i: (0, 0)),
            pl.BlockSpec((1, 128), lambda i: (0, 0)),
        ],
        out_specs=pl.BlockSpec((BN, 128), lambda i: (i, 0)),
        out_shape=jax.ShapeDtypeStruct((N, 128), _f32),
    )(h2, W1, b1.reshape(1, 128), W2p, b2p.reshape(1, 128))


from jax.experimental.pallas import tpu_sc as plsc

NW = 32
CH = 40


def _mesh():
    return plsc.VectorSubcoreMesh(core_axis_name="c", subcore_axis_name="s")


def _gather_qk(q, k, dst, src):
    per = E // NW
    CHG = 128
    nch = per // CHG
    TL = per - nch * CHG
    @functools.partial(
        pl.kernel,
        out_type=[jax.ShapeDtypeStruct((E, D), _f32)] * 2,
        mesh=_mesh(),
        scratch_types=[
            pltpu.VMEM((CHG,), jnp.int32),
            pltpu.VMEM((CHG,), jnp.int32),
            pltpu.VMEM((CHG, D), _f32),
            pltpu.VMEM((CHG, D), _f32),
            pltpu.SemaphoreType.DMA,
            pltpu.SemaphoreType.DMA,
        ],
    )
    def run(q_hbm, k_hbm, dst_hbm, src_hbm, qd_hbm, ks_hbm,
            idxd, idxs, qrows, krows, sem1, sem2):
        wid = lax.axis_index("s") * 2 + lax.axis_index("c")
        base0 = wid * per

        def chunk(base, n):
            ci = pltpu.async_copy(dst_hbm.at[pl.ds(base, n)],
                                  idxd.at[pl.ds(0, n)], sem1)
            cj = pltpu.async_copy(src_hbm.at[pl.ds(base, n)],
                                  idxs.at[pl.ds(0, n)], sem2)
            ci.wait()
            cj.wait()
            cq = pltpu.async_copy(q_hbm.at[idxd.at[pl.ds(0, n)]],
                                  qrows.at[pl.ds(0, n)], sem1)
            ck = pltpu.async_copy(k_hbm.at[idxs.at[pl.ds(0, n)]],
                                  krows.at[pl.ds(0, n)], sem2)
            cq.wait()
            ck.wait()
            wq = pltpu.async_copy(qrows.at[pl.ds(0, n)],
                                  qd_hbm.at[pl.ds(base, n)], sem1)
            wk = pltpu.async_copy(krows.at[pl.ds(0, n)],
                                  ks_hbm.at[pl.ds(base, n)], sem2)
            wq.wait()
            wk.wait()

        def step(i, carry):
            chunk(base0 + i * CHG, CHG)
            return carry

        lax.fori_loop(0, nch, step, 0)
        chunk(base0 + nch * CHG, TL)

    return run(q, k, dst, src)


def _gather_v(v, src):
    per = E // NW
    CHG = 128
    nch = per // CHG
    TL = per - nch * CHG
    npair = nch // 2

    @functools.partial(
        pl.kernel,
        out_type=jax.ShapeDtypeStruct((E, D), _f32),
        mesh=_mesh(),
        scratch_types=[
            pltpu.VMEM((CHG,), jnp.int32),
            pltpu.VMEM((CHG,), jnp.int32),
            pltpu.VMEM((CHG, D), _f32),
            pltpu.VMEM((CHG, D), _f32),
            pltpu.SemaphoreType.DMA,
            pltpu.SemaphoreType.DMA,
        ],
    )
    def run(v_hbm, src_hbm, vs_hbm, idx0, idx1, r0, r1, sem1, sem2):
        wid = lax.axis_index("s") * 2 + lax.axis_index("c")
        base0 = wid * per

        def half(base, n, idxb, rb, sem):
            ci = pltpu.async_copy(src_hbm.at[pl.ds(base, n)],
                                  idxb.at[pl.ds(0, n)], sem)
            ci.wait()
            cg = pltpu.async_copy(v_hbm.at[idxb.at[pl.ds(0, n)]],
                                  rb.at[pl.ds(0, n)], sem)
            return cg

        def step(i, carry):
            b = base0 + i * 2 * CHG
            c0 = half(b, CHG, idx0, r0, sem1)
            c1 = half(b + CHG, CHG, idx1, r1, sem2)
            c0.wait()
            w0 = pltpu.async_copy(r0, vs_hbm.at[pl.ds(b, CHG)], sem1)
            c1.wait()
            w1 = pltpu.async_copy(r1, vs_hbm.at[pl.ds(b + CHG, CHG)], sem2)
            w0.wait()
            w1.wait()
            return carry

        lax.fori_loop(0, npair, step, 0)
        b = base0 + npair * 2 * CHG
        c0 = half(b, CHG, idx0, r0, sem1)
        c1 = half(b + CHG, TL, idx1, r1, sem2)
        c0.wait()
        pltpu.async_copy(r0, vs_hbm.at[pl.ds(b, CHG)], sem1).wait()
        c1.wait()
        pltpu.async_copy(r1.at[pl.ds(0, TL)],
                         vs_hbm.at[pl.ds(b + CHG, TL)], sem2).wait()

    return run(v, src)


NP = 10240


def _scatter_den(ex, dst):
    per = (E // 2) // 16
    CHS = 128
    G = 2
    NR = NP // 16
    NF = 10
    FR = NR // NF

    @functools.partial(
        pl.kernel,
        out_type=jax.ShapeDtypeStruct((2, NP, 128), _f32),
        mesh=_mesh(),
        scratch_types=(
            [pltpu.VMEM_SHARED((NP, 128), _f32)]
            + [pltpu.VMEM((CHS,), jnp.int32)] * G
            + [pltpu.VMEM((CHS, 128), _f32)] * G
            + [pltpu.VMEM((FR, 128), _f32)]
            + [pltpu.SemaphoreType.DMA]
        ),
    )
    def run(ex_hbm, dst_hbm, den_hbm, acc,
            i0, i1, m0, m1,
            fbuf, semI):
        semM = semI
        c = lax.axis_index("c")
        sid = lax.axis_index("s")
        idxb = [i0, i1]
        mbuf = [m0, m1]

        def zrow(j, carry):
            for t in range(8):
                fbuf[j, pl.ds(t * 16, 16)] = jnp.zeros((16,), _f32)
            return carry

        lax.fori_loop(0, FR, zrow, 0)
        for f in range(NF):
            pltpu.sync_copy(fbuf, acc.at[pl.ds(sid * NR + f * FR, FR)])
        plsc.subcore_barrier()

        base0 = c * (E // 2) + sid * per

        def burst(gb, k):
            descs = []
            for j in range(k):
                b = gb + j * CHS
                descs.append(pltpu.async_copy(
                    dst_hbm.at[pl.ds(b, CHS)], idxb[j], semI))
                descs.append(pltpu.async_copy(
                    ex_hbm.at[pl.ds(b, CHS)], mbuf[j], semM))
            for d in descs:
                d.wait()
            for j in range(k):
                pltpu.sync_copy(mbuf[j], acc.at[idxb[j]], add=True)

        nch = per // CHS
        ngrp = nch // G
        rem = nch - ngrp * G
        tl = per - nch * CHS

        def group(i, carry):
            burst(base0 + i * G * CHS, G)
            return carry

        lax.fori_loop(0, ngrp, group, 0)
        burst(base0 + ngrp * G * CHS, rem)
        tb = base0 + nch * CHS
        pltpu.sync_copy(dst_hbm.at[pl.ds(tb, tl)], i0.at[pl.ds(0, tl)])
        pltpu.sync_copy(ex_hbm.at[pl.ds(tb, tl)], m0.at[pl.ds(0, tl)])
        pltpu.sync_copy(m0.at[pl.ds(0, tl)],
                        acc.at[i0.at[pl.ds(0, tl)]], add=True)

        plsc.subcore_barrier()
        for f in range(NF):
            pltpu.sync_copy(acc.at[pl.ds(sid * NR + f * FR, FR)], fbuf)
            pltpu.sync_copy(fbuf, den_hbm.at[c, pl.ds(sid * NR + f * FR, FR)])

    return run(ex, dst)


def _scatter_msg(ma, mb, mc, dst):
    CHS = 128
    G = 2
    per_full = E // 16
    per_half = (E // 2) // 16
    NR = NP // 16
    NF = 10
    FR = NR // NF

    @functools.partial(
        pl.kernel,
        out_type=[jax.ShapeDtypeStruct((2, NP, 128), _f32)] * 2,
        mesh=_mesh(),
        scratch_types=(
            [pltpu.VMEM_SHARED((NP, 128), _f32)]
            + [pltpu.VMEM((CHS,), jnp.int32)] * G
            + [pltpu.VMEM((CHS, 128), _f32)] * G
            + [pltpu.VMEM((FR, 128), _f32)]
            + [pltpu.SemaphoreType.DMA]
        ),
    )
    def run(ma_hbm, mb_hbm, mc_hbm, dst_hbm, o1_hbm, o2_hbm, acc,
            i0, i1, m0, m1,
            fbuf, semI):
        semM = semI
        c = lax.axis_index("c")
        sid = lax.axis_index("s")
        idxb = [i0, i1]
        mbuf = [m0, m1]

        def zrow(j, carry):
            for t in range(8):
                fbuf[j, pl.ds(t * 16, 16)] = jnp.zeros((16,), _f32)
            return carry

        def zero_acc():
            lax.fori_loop(0, FR, zrow, 0)
            for f in range(NF):
                pltpu.sync_copy(fbuf, acc.at[pl.ds(sid * NR + f * FR, FR)])

        def accumulate(m_hbm, base0, n_edges):
            def burst(gb, k):
                descs = []
                for j in range(k):
                    b = gb + j * CHS
                    descs.append(pltpu.async_copy(
                        dst_hbm.at[pl.ds(b, CHS)], idxb[j], semI))
                    descs.append(pltpu.async_copy(
                        m_hbm.at[pl.ds(b, CHS)], mbuf[j], semM))
                for d in descs:
                    d.wait()
                for j in range(k):
                    pltpu.sync_copy(mbuf[j], acc.at[idxb[j]], add=True)

            nch = n_edges // CHS
            ngrp = nch // G
            rem = nch - ngrp * G
            tl = n_edges - nch * CHS

            def group(i, carry):
                burst(base0 + i * G * CHS, G)
                return carry

            lax.fori_loop(0, ngrp, group, 0)
            if rem:
                burst(base0 + ngrp * G * CHS, rem)
            tb = base0 + nch * CHS
            pltpu.sync_copy(dst_hbm.at[pl.ds(tb, tl)], i0.at[pl.ds(0, tl)])
            pltpu.sync_copy(m_hbm.at[pl.ds(tb, tl)], m0.at[pl.ds(0, tl)])
            pltpu.sync_copy(m0.at[pl.ds(0, tl)],
                            acc.at[i0.at[pl.ds(0, tl)]], add=True)

        def flush(o_hbm):
            for f in range(NF):
                pltpu.sync_copy(acc.at[pl.ds(sid * NR + f * FR, FR)], fbuf)
                pltpu.sync_copy(fbuf, o_hbm.at[c, pl.ds(sid * NR + f * FR, FR)])

        zero_acc()
        plsc.subcore_barrier()

        @pl.when(c == 0)
        def _():
            accumulate(ma_hbm, sid * per_full, per_full)

        @pl.when(c == 1)
        def _():
            accumulate(mb_hbm, sid * per_full, per_full)

        plsc.subcore_barrier()
        flush(o1_hbm)
        plsc.subcore_barrier()

        zero_acc()
        plsc.subcore_barrier()
        accumulate(mc_hbm, c * (E // 2) + sid * per_half, per_half)
        plsc.subcore_barrier()
        flush(o2_hbm)

    return run(ma, mb, mc, dst)


def _gln_scales(st_row, count, w, b):
    m = st_row[0] / count
    var = st_row[1] / count - m * m
    sd = jnp.sqrt(jnp.maximum(var, 0.0))
    inv = 1.0 / (sd + EPS)
    return w * inv, b - m * w * inv


def kernel(x_graph, x_visual, x_prior, edge_index, edge_attr, batch, params):
    p = params
    src = edge_index[0]
    dst = edge_index[1]

    z_node, st_n = _encode_nodes(x_visual, x_graph, x_prior, p)
    ze, st_e = _encode_edges(edge_attr, p)

    cnt_n = float(N * 128)
    wv, bv = _gln_scales(st_n[0, 0:2], cnt_n, p['lnv_w'], p['lnv_b'])
    wg, bg = _gln_scales(st_n[0, 2:4], cnt_n, p['lng_w'], p['lng_b'])
    wp_, bp_ = _gln_scales(st_n[0, 4:6], cnt_n, p['lnp_w'], p['lnp_b'])
    wcat = jnp.concatenate([wv, wg, wp_]).reshape(1, D)
    bcat = jnp.concatenate([bv, bg, bp_]).reshape(1, D)
    h0 = _affine(z_node, wcat, bcat)

    we, be = _gln_scales(st_e[0, 0:2], float(E * 128), p['lne_w'], p['lne_b'])
    W1p = we[:, None] * p['Wedge1']
    b1p = be @ p['Wedge1']
    W2p = we[:, None] * p['Wedge2']
    b2p = be @ p['Wedge2']
    e1, e2 = _edgeproj(ze, W1p, b1p, W2p, b2p)

    hsel128 = (jnp.arange(D)[:, None] // C == jnp.arange(128)[None, :]).astype(_f32)
    hd = hsel128.T
    oh = (batch[:, None] == jnp.arange(128)[None, :]).astype(_f32)

    h = h0
    for s, e_l in (('1', e1), ('2', e2)):
        q, k, v = _qkv(h, p, s)
        qd, ks = _gather_qk(q, k, dst, src)
        vs = _gather_v(v, src)
        ex, ma, mb, mc = _edge_attn(qd, ks, vs, e_l, hsel128, hd)
        den_parts = _scatter_den(ex, dst)
        msg_full, msg_parts = _scatter_msg(ma, mb, mc, dst)
        t, st_g = _combine(h, msg_full, msg_parts, den_parts, hd,
                           p['Wskip' + s], p['bskip' + s], oh)
        cnt = jnp.maximum(st_g[:, 2], 1.0)
        mean = st_g[:, 0] / cnt
        var = st_g[:, 1] / cnt - mean * mean
        r = 1.0 / (jnp.sqrt(jnp.maximum(var, 0.0)) + EPS)
        mv = jnp.stack([mean, r], axis=1)
        h = _bnorm(t, oh, mv, p['ln' + s + '_w'], p['ln' + s + '_b'])

    Wc2p = jnp.pad(p['Wc2'], ((0, 0), (0, 128 - 49)))
    bc2p = jnp.pad(p['bc2'], (0, 128 - 49))
    logits = _final(h, p['Wc1'], p['bc1'], Wc2p, bc2p)
    return logits[:, :49]
